# baseline (speedup 1.0000x reference)
import jax
import jax.numpy as jnp
from jax.experimental import pallas as pl
from jax.experimental.pallas import tpu as pltpu

B, N, D = 4096, 512, 16
VEL_W, SUR_W = 0.6, 0.4
BR = 32
STEPS = B // BR
RWS = BR * N
SBR = 256
SSTEPS = B // SBR


def _salience_body(zt_any, zt_ref, ztm_ref, p_ref, out_any, idx_ref,
                   acc_ref, pt_ref, ny_ref, sem):
    i = pl.program_id(0)

    @pl.when(i == 0)
    def _start_copy():
        pltpu.make_async_copy(zt_any, out_any, sem).start()

    zt = zt_ref[...]
    ztm = ztm_ref[...]

    ztT = zt.reshape(RWS, D).T
    ztmT = ztm.reshape(RWS, D).T

    @pl.when(i == 0)
    def _prep():
        pT = p_ref[...].T
        pt_ref[...] = jnp.tile(pT, (1, BR))
        ny_ref[...] = jnp.sqrt(
            jnp.sum(pT * pT, axis=0, keepdims=True))

    ptT = pt_ref[...]

    diffT = ztT - ztmT
    vel2 = jnp.sum(diffT * diffT, axis=0, keepdims=True)
    zdot = jnp.sum(ztT * ptT, axis=0, keepdims=True)
    nx2 = jnp.sum(ztT * ztT, axis=0, keepdims=True)

    vel2 = vel2.reshape(BR, N)
    zdot = zdot.reshape(BR, N)
    nx2 = nx2.reshape(BR, N)

    vel = jnp.sqrt(vel2)
    nx = jnp.sqrt(nx2)
    ny = ny_ref[...]
    cos = zdot / jnp.maximum(nx * ny, 1e-8)
    surprise = jnp.clip(1.0 - cos, 0.0, 2.0) / 2.0
    sal = VEL_W * vel + SUR_W * surprise

    mn = jnp.min(sal, axis=-1, keepdims=True)
    mx = jnp.max(sal, axis=-1, keepdims=True)
    saln = (sal - mn) / (mx - mn + 1e-8)
    bsum = jnp.sum(saln, axis=0, keepdims=True)

    @pl.when(i == 0)
    def _init():
        acc_ref[...] = bsum

    @pl.when(i != 0)
    def _accum():
        acc_ref[...] = acc_ref[...] + bsum

    @pl.when(i == STEPS - 1)
    def _finish():
        acc = acc_ref[...]
        m = jnp.max(acc)
        eid = jax.lax.broadcasted_iota(jnp.int32, (1, N), 1)
        idx_ref[0, 0] = jnp.min(jnp.where(acc == m, eid, jnp.int32(2**30)))
        pltpu.make_async_copy(zt_any, out_any, sem).wait()


def _scatter_body(idx_ref, mt_ref, y_ref, o_ref):
    sub = idx_ref[0] % 8
    ent = jax.lax.broadcasted_iota(jnp.int32, (SBR, 8, D), 1)
    o_ref[...] = jnp.where(ent == sub, mt_ref[...], y_ref[...])


def kernel(z_t, z_tm1, prior, mask_token):
    out_copy, idx = pl.pallas_call(
        _salience_body,
        grid=(STEPS,),
        in_specs=[
            pl.BlockSpec(memory_space=pl.ANY),
            pl.BlockSpec((BR, N, D), lambda i: (i, 0, 0)),
            pl.BlockSpec((BR, N, D), lambda i: (i, 0, 0)),
            pl.BlockSpec((N, D), lambda i: (0, 0)),
        ],
        out_specs=[
            pl.BlockSpec(memory_space=pl.ANY),
            pl.BlockSpec(memory_space=pltpu.SMEM),
        ],
        out_shape=[
            jax.ShapeDtypeStruct((B, N, D), jnp.float32),
            jax.ShapeDtypeStruct((1, 1), jnp.int32),
        ],
        scratch_shapes=[
            pltpu.VMEM((1, N), jnp.float32),
            pltpu.VMEM((D, RWS), jnp.float32),
            pltpu.VMEM((1, N), jnp.float32),
            pltpu.SemaphoreType.DMA,
        ],
    )(z_t, z_t, z_tm1, prior)

    mt3 = mask_token.reshape(1, 1, D)
    idx_flat = idx.reshape((1,))

    masked = pl.pallas_call(
        _scatter_body,
        grid_spec=pltpu.PrefetchScalarGridSpec(
            num_scalar_prefetch=1,
            grid=(SSTEPS,),
            in_specs=[
                pl.BlockSpec((1, 1, D), lambda i, sref: (0, 0, 0)),
                pl.BlockSpec((SBR, 8, D), lambda i, sref: (i, sref[0] // 8, 0)),
            ],
            out_specs=pl.BlockSpec((SBR, 8, D), lambda i, sref: (i, sref[0] // 8, 0)),
        ),
        out_shape=jax.ShapeDtypeStruct((B, N, D), jnp.float32),
        input_output_aliases={2: 0},
    )(idx_flat, mt3, out_copy)

    return masked


# speedup vs baseline: 12.8267x; 12.8267x over previous
import jax
import jax.numpy as jnp
from jax.experimental import pallas as pl
from jax.experimental.pallas import tpu as pltpu

B, N, D = 4096, 512, 16
VEL_W, SUR_W = 0.6, 0.4
BR = 32
STEPS = B // BR
RWS = BR * N
SBR = 256
SSTEPS = B // SBR


def _salience_body(zt_ref, ztm_ref, p_ref, out_ref, idx_ref,
                   acc_ref, pt_ref, ny_ref):
    i = pl.program_id(0)
    zt = zt_ref[...]
    out_ref[...] = zt
    ztm = ztm_ref[...]

    ztT = zt.reshape(RWS, D).T
    ztmT = ztm.reshape(RWS, D).T

    @pl.when(i == 0)
    def _prep():
        pT = p_ref[...].T
        pt_ref[...] = jnp.tile(pT, (1, BR))
        ny_ref[...] = jnp.sqrt(
            jnp.sum(pT * pT, axis=0, keepdims=True))

    ptT = pt_ref[...]

    diffT = ztT - ztmT
    vel2 = jnp.sum(diffT * diffT, axis=0, keepdims=True)
    zdot = jnp.sum(ztT * ptT, axis=0, keepdims=True)
    nx2 = jnp.sum(ztT * ztT, axis=0, keepdims=True)

    vel2 = vel2.reshape(BR, N)
    zdot = zdot.reshape(BR, N)
    nx2 = nx2.reshape(BR, N)

    vel = jnp.sqrt(vel2)
    nx = jnp.sqrt(nx2)
    ny = ny_ref[...]
    cos = zdot / jnp.maximum(nx * ny, 1e-8)
    surprise = jnp.clip(1.0 - cos, 0.0, 2.0) / 2.0
    sal = VEL_W * vel + SUR_W * surprise

    mn = jnp.min(sal, axis=-1, keepdims=True)
    mx = jnp.max(sal, axis=-1, keepdims=True)
    saln = (sal - mn) / (mx - mn + 1e-8)
    bsum = jnp.sum(saln, axis=0, keepdims=True)

    @pl.when(i == 0)
    def _init():
        acc_ref[...] = bsum

    @pl.when(i != 0)
    def _accum():
        acc_ref[...] = acc_ref[...] + bsum

    @pl.when(i == STEPS - 1)
    def _finish():
        acc = acc_ref[...]
        m = jnp.max(acc)
        eid = jax.lax.broadcasted_iota(jnp.int32, (1, N), 1)
        idx_ref[0, 0] = jnp.min(jnp.where(acc == m, eid, jnp.int32(2**30)))


def _scatter_body(idx_ref, mt_ref, y_ref, o_ref):
    sub = idx_ref[0] % 8
    ent = jax.lax.broadcasted_iota(jnp.int32, (SBR, 8, D), 1)
    o_ref[...] = jnp.where(ent == sub, mt_ref[...], y_ref[...])


def kernel(z_t, z_tm1, prior, mask_token):
    out_copy, idx = pl.pallas_call(
        _salience_body,
        grid=(STEPS,),
        in_specs=[
            pl.BlockSpec((BR, N, D), lambda i: (i, 0, 0)),
            pl.BlockSpec((BR, N, D), lambda i: (i, 0, 0)),
            pl.BlockSpec((N, D), lambda i: (0, 0)),
        ],
        out_specs=[
            pl.BlockSpec((BR, N, D), lambda i: (i, 0, 0)),
            pl.BlockSpec(memory_space=pltpu.SMEM),
        ],
        out_shape=[
            jax.ShapeDtypeStruct((B, N, D), jnp.float32),
            jax.ShapeDtypeStruct((1, 1), jnp.int32),
        ],
        scratch_shapes=[
            pltpu.VMEM((1, N), jnp.float32),
            pltpu.VMEM((D, RWS), jnp.float32),
            pltpu.VMEM((1, N), jnp.float32),
        ],
    )(z_t, z_tm1, prior)

    mt3 = mask_token.reshape(1, 1, D)
    idx_flat = idx.reshape((1,))

    masked = pl.pallas_call(
        _scatter_body,
        grid_spec=pltpu.PrefetchScalarGridSpec(
            num_scalar_prefetch=1,
            grid=(SSTEPS,),
            in_specs=[
                pl.BlockSpec((1, 1, D), lambda i, sref: (0, 0, 0)),
                pl.BlockSpec((SBR, 8, D), lambda i, sref: (i, sref[0] // 8, 0)),
            ],
            out_specs=pl.BlockSpec((SBR, 8, D), lambda i, sref: (i, sref[0] // 8, 0)),
        ),
        out_shape=jax.ShapeDtypeStruct((B, N, D), jnp.float32),
        input_output_aliases={2: 0},
    )(idx_flat, mt3, out_copy)

    return masked


# speedup vs baseline: 14.8189x; 1.1553x over previous
import jax
import jax.numpy as jnp
from jax.experimental import pallas as pl
from jax.experimental.pallas import tpu as pltpu

B, N, D = 4096, 512, 16
VEL_W, SUR_W = 0.6, 0.4
GPR = N // 8
BR = 64
RG = BR * GPR
STEPS = B // BR


def _segsum16(x):
    xt = x.T
    for sh in (1, 2, 4, 8):
        xt = xt + pltpu.roll(xt, 128 - sh, 0)
    sums = xt.reshape(8, 16, x.shape[0])[:, 0, :]
    return sums.T


def _salience_body(zt_ref, ztm_ref, pt_ref, out_ref, idx_ref, acc_ref):
    i = pl.program_id(0)
    zt = zt_ref[...]
    out_ref[...] = zt
    ztm = ztm_ref[...]
    ptile = pt_ref[...]

    diff = zt - ztm
    vel2 = _segsum16(diff * diff)
    zdot = _segsum16(zt * ptile)
    nx2 = _segsum16(zt * zt)
    ny2 = _segsum16(ptile * ptile)

    vel = jnp.sqrt(vel2)
    nx = jnp.sqrt(nx2)
    ny = jnp.sqrt(ny2)
    cos = zdot / jnp.maximum(nx * ny, 1e-8)
    surprise = jnp.clip(1.0 - cos, 0.0, 2.0) / 2.0
    sal = (VEL_W * vel + SUR_W * surprise).reshape(BR, GPR, 8)

    mn = jnp.min(sal, axis=(1, 2), keepdims=True)
    mx = jnp.max(sal, axis=(1, 2), keepdims=True)
    saln = (sal - mn) / (mx - mn + 1e-8)
    bsum = jnp.sum(saln, axis=0)

    @pl.when(i == 0)
    def _init():
        acc_ref[...] = bsum

    @pl.when(i != 0)
    def _accum():
        acc_ref[...] = acc_ref[...] + bsum

    @pl.when(i == STEPS - 1)
    def _finish():
        acc = acc_ref[...]
        m = jnp.max(acc)
        eid = (jax.lax.broadcasted_iota(jnp.int32, (GPR, 8), 0) * 8
               + jax.lax.broadcasted_iota(jnp.int32, (GPR, 8), 1))
        idx_ref[0, 0] = jnp.min(jnp.where(acc == m, eid, jnp.int32(2**30)))


def _scatter_body(idx_ref, mt_ref, y_ref, o_ref):
    off = (idx_ref[0] % 8) * D
    lane = jax.lax.broadcasted_iota(jnp.int32, (B, 128), 1)
    sel = (lane >= off) & (lane < off + D)
    o_ref[...] = jnp.where(sel, mt_ref[...], y_ref[...])


def kernel(z_t, z_tm1, prior, mask_token):
    z2 = z_t.reshape(B * GPR, 128)
    zm2 = z_tm1.reshape(B * GPR, 128)
    p2 = prior.reshape(GPR, 128)
    ptile = jnp.tile(p2, (BR, 1))

    out_copy, idx = pl.pallas_call(
        _salience_body,
        grid=(STEPS,),
        in_specs=[
            pl.BlockSpec((RG, 128), lambda i: (i, 0)),
            pl.BlockSpec((RG, 128), lambda i: (i, 0)),
            pl.BlockSpec((RG, 128), lambda i: (0, 0)),
        ],
        out_specs=[
            pl.BlockSpec((RG, 128), lambda i: (i, 0)),
            pl.BlockSpec(memory_space=pltpu.SMEM),
        ],
        out_shape=[
            jax.ShapeDtypeStruct((B * GPR, 128), jnp.float32),
            jax.ShapeDtypeStruct((1, 1), jnp.int32),
        ],
        scratch_shapes=[pltpu.VMEM((GPR, 8), jnp.float32)],
    )(z2, zm2, ptile)

    y = out_copy.reshape(B, N * D)
    mt2 = jnp.tile(mask_token.reshape(1, D), (1, 8))
    idx_flat = idx.reshape((1,))

    masked = pl.pallas_call(
        _scatter_body,
        grid_spec=pltpu.PrefetchScalarGridSpec(
            num_scalar_prefetch=1,
            grid=(1,),
            in_specs=[
                pl.BlockSpec((1, 128), lambda i, sref: (0, 0)),
                pl.BlockSpec((B, 128), lambda i, sref: (0, sref[0] // 8)),
            ],
            out_specs=pl.BlockSpec((B, 128), lambda i, sref: (0, sref[0] // 8)),
        ),
        out_shape=jax.ShapeDtypeStruct((B, N * D), jnp.float32),
        input_output_aliases={2: 0},
    )(idx_flat, mt2, y)

    return masked.reshape(B, N, D)


# speedup vs baseline: 15.1792x; 1.0243x over previous
import jax
import jax.numpy as jnp
from jax.experimental import pallas as pl
from jax.experimental.pallas import tpu as pltpu

B, N, D = 4096, 512, 16
VEL_W, SUR_W = 0.6, 0.4
GPR = N // 8
BR = 64
RG = BR * GPR
STEPS = B // BR


def _segsum16(x):
    xt = x.T
    for sh in (1, 2, 4, 8):
        xt = xt + pltpu.roll(xt, 128 - sh, 0)
    sums = xt.reshape(8, 16, x.shape[0])[:, 0, :]
    return sums.T


def _salience_body(zt_ref, ztm_ref, pt_ref, out_ref, idx_ref,
                   acc_ref, ny_ref):
    i = pl.program_id(0)
    zt = zt_ref[...]
    out_ref[...] = zt
    ztm = ztm_ref[...]
    ptile = pt_ref[...]

    @pl.when(i == 0)
    def _prep():
        ny_ref[...] = jnp.sqrt(_segsum16(ptile * ptile))

    diff = zt - ztm
    vel2 = _segsum16(diff * diff)
    zdot = _segsum16(zt * ptile)
    nx2 = _segsum16(zt * zt)

    vel = jnp.sqrt(vel2)
    nx = jnp.sqrt(nx2)
    ny = ny_ref[...]
    cos = zdot / jnp.maximum(nx * ny, 1e-8)
    surprise = jnp.clip(1.0 - cos, 0.0, 2.0) / 2.0
    sal = (VEL_W * vel + SUR_W * surprise).reshape(BR, GPR, 8)

    mn = jnp.min(sal, axis=(1, 2), keepdims=True)
    mx = jnp.max(sal, axis=(1, 2), keepdims=True)
    saln = (sal - mn) / (mx - mn + 1e-8)
    bsum = jnp.sum(saln, axis=0)

    @pl.when(i == 0)
    def _init():
        acc_ref[...] = bsum

    @pl.when(i != 0)
    def _accum():
        acc_ref[...] = acc_ref[...] + bsum

    @pl.when(i == STEPS - 1)
    def _finish():
        acc = acc_ref[...]
        m = jnp.max(acc)
        eid = (jax.lax.broadcasted_iota(jnp.int32, (GPR, 8), 0) * 8
               + jax.lax.broadcasted_iota(jnp.int32, (GPR, 8), 1))
        idx_ref[0, 0] = jnp.min(jnp.where(acc == m, eid, jnp.int32(2**30)))


def _scatter_body(idx_ref, mt_ref, y_ref, o_ref):
    off = (idx_ref[0] % 8) * D
    lane = jax.lax.broadcasted_iota(jnp.int32, (B, 128), 1)
    sel = (lane >= off) & (lane < off + D)
    o_ref[...] = jnp.where(sel, mt_ref[...], y_ref[...])


def kernel(z_t, z_tm1, prior, mask_token):
    z2 = z_t.reshape(B * GPR, 128)
    zm2 = z_tm1.reshape(B * GPR, 128)
    p2 = prior.reshape(GPR, 128)
    ptile = jnp.tile(p2, (BR, 1))

    out_copy, idx = pl.pallas_call(
        _salience_body,
        grid=(STEPS,),
        in_specs=[
            pl.BlockSpec((RG, 128), lambda i: (i, 0)),
            pl.BlockSpec((RG, 128), lambda i: (i, 0)),
            pl.BlockSpec((RG, 128), lambda i: (0, 0)),
        ],
        out_specs=[
            pl.BlockSpec((RG, 128), lambda i: (i, 0)),
            pl.BlockSpec(memory_space=pltpu.SMEM),
        ],
        out_shape=[
            jax.ShapeDtypeStruct((B * GPR, 128), jnp.float32),
            jax.ShapeDtypeStruct((1, 1), jnp.int32),
        ],
        scratch_shapes=[
            pltpu.VMEM((GPR, 8), jnp.float32),
            pltpu.VMEM((RG, 8), jnp.float32),
        ],
    )(z2, zm2, ptile)

    y = out_copy.reshape(B, N * D)
    mt2 = jnp.tile(mask_token.reshape(1, D), (1, 8))
    idx_flat = idx.reshape((1,))

    masked = pl.pallas_call(
        _scatter_body,
        grid_spec=pltpu.PrefetchScalarGridSpec(
            num_scalar_prefetch=1,
            grid=(1,),
            in_specs=[
                pl.BlockSpec((1, 128), lambda i, sref: (0, 0)),
                pl.BlockSpec((B, 128), lambda i, sref: (0, sref[0] // 8)),
            ],
            out_specs=pl.BlockSpec((B, 128), lambda i, sref: (0, sref[0] // 8)),
        ),
        out_shape=jax.ShapeDtypeStruct((B, N * D), jnp.float32),
        input_output_aliases={2: 0},
    )(idx_flat, mt2, y)

    return masked.reshape(B, N, D)


# speedup vs baseline: 18.2662x; 1.2034x over previous
import jax
import jax.numpy as jnp
from jax.experimental import pallas as pl
from jax.experimental.pallas import tpu as pltpu

B, N, D = 4096, 512, 16
VEL_W, SUR_W = 0.6, 0.4
BR = 32
STEPS = B // BR
RWS = BR * N
SBR = 256
SSTEPS = B // SBR


def _salience_body(zt_ref, ztm_ref, p_ref, out_ref, idx_ref,
                   acc_ref, pt_ref, ny_ref):
    i = pl.program_id(0)
    zt = zt_ref[...]
    out_ref[...] = zt
    ztm = ztm_ref[...]

    ztT = zt.T
    ztmT = ztm.T

    @pl.when(i == 0)
    def _prep():
        pT = p_ref[...].T
        pt_ref[...] = jnp.tile(pT, (1, BR))
        ny_ref[...] = jnp.sqrt(
            jnp.sum(pT * pT, axis=0, keepdims=True))

    ptT = pt_ref[...]

    diffT = ztT - ztmT
    vel2 = jnp.sum(diffT * diffT, axis=0, keepdims=True)
    zdot = jnp.sum(ztT * ptT, axis=0, keepdims=True)
    nx2 = jnp.sum(ztT * ztT, axis=0, keepdims=True)

    vel2 = vel2.reshape(BR, N)
    zdot = zdot.reshape(BR, N)
    nx2 = nx2.reshape(BR, N)

    vel = jnp.sqrt(vel2)
    nx = jnp.sqrt(nx2)
    ny = ny_ref[...]
    cos = zdot / jnp.maximum(nx * ny, 1e-8)
    surprise = jnp.clip(1.0 - cos, 0.0, 2.0) / 2.0
    sal = VEL_W * vel + SUR_W * surprise

    mn = jnp.min(sal, axis=-1, keepdims=True)
    mx = jnp.max(sal, axis=-1, keepdims=True)
    saln = (sal - mn) / (mx - mn + 1e-8)
    bsum = jnp.sum(saln, axis=0, keepdims=True)

    @pl.when(i == 0)
    def _init():
        acc_ref[...] = bsum

    @pl.when(i != 0)
    def _accum():
        acc_ref[...] = acc_ref[...] + bsum

    @pl.when(i == STEPS - 1)
    def _finish():
        acc = acc_ref[...]
        m = jnp.max(acc)
        eid = jax.lax.broadcasted_iota(jnp.int32, (1, N), 1)
        idx_ref[0, 0] = jnp.min(jnp.where(acc == m, eid, jnp.int32(2**30)))


def _scatter_body(idx_ref, mt_ref, y_ref, o_ref):
    sub = idx_ref[0] % 8
    ent = jax.lax.broadcasted_iota(jnp.int32, (SBR, 8, D), 1)
    o_ref[...] = jnp.where(ent == sub, mt_ref[...], y_ref[...])


def kernel(z_t, z_tm1, prior, mask_token):
    z2 = z_t.reshape(B * N, D)
    zm2 = z_tm1.reshape(B * N, D)

    out_copy, idx = pl.pallas_call(
        _salience_body,
        grid=(STEPS,),
        in_specs=[
            pl.BlockSpec((RWS, D), lambda i: (i, 0)),
            pl.BlockSpec((RWS, D), lambda i: (i, 0)),
            pl.BlockSpec((N, D), lambda i: (0, 0)),
        ],
        out_specs=[
            pl.BlockSpec((RWS, D), lambda i: (i, 0)),
            pl.BlockSpec(memory_space=pltpu.SMEM),
        ],
        out_shape=[
            jax.ShapeDtypeStruct((B * N, D), jnp.float32),
            jax.ShapeDtypeStruct((1, 1), jnp.int32),
        ],
        scratch_shapes=[
            pltpu.VMEM((1, N), jnp.float32),
            pltpu.VMEM((D, RWS), jnp.float32),
            pltpu.VMEM((1, N), jnp.float32),
        ],
    )(z2, zm2, prior)

    y = out_copy.reshape(B, N, D)
    mt3 = mask_token.reshape(1, 1, D)
    idx_flat = idx.reshape((1,))

    masked = pl.pallas_call(
        _scatter_body,
        grid_spec=pltpu.PrefetchScalarGridSpec(
            num_scalar_prefetch=1,
            grid=(SSTEPS,),
            in_specs=[
                pl.BlockSpec((1, 1, D), lambda i, sref: (0, 0, 0)),
                pl.BlockSpec((SBR, 8, D), lambda i, sref: (i, sref[0] // 8, 0)),
            ],
            out_specs=pl.BlockSpec((SBR, 8, D), lambda i, sref: (i, sref[0] // 8, 0)),
        ),
        out_shape=jax.ShapeDtypeStruct((B, N, D), jnp.float32),
        input_output_aliases={2: 0},
    )(idx_flat, mt3, y)

    return masked


# speedup vs baseline: 23.5741x; 1.2906x over previous
import jax
import jax.numpy as jnp
from jax.experimental import pallas as pl
from jax.experimental.pallas import tpu as pltpu

B, N, D = 4096, 512, 16
VEL_W, SUR_W = 0.6, 0.4
BR = 32
STEPS = B // BR
RWS = BR * N
SBR = 256
SSTEPS = B // SBR


def _salience_body(zt_ref, ztm_ref, p_ref, out_ref, idx_ref,
                   acc_ref, pt_ref, ny_ref):
    i = pl.program_id(0)
    zt = zt_ref[...]
    out_ref[...] = zt
    ztm = ztm_ref[...]

    ztT = zt.T
    ztmT = ztm.T

    @pl.when(i == 0)
    def _prep():
        pT = p_ref[...].T
        pt_ref[...] = jnp.tile(pT, (1, BR))
        ny_ref[...] = jnp.sqrt(
            jnp.sum(pT * pT, axis=0, keepdims=True))

    ptT = pt_ref[...]

    diffT = ztT - ztmT
    vel2 = jnp.sum(diffT * diffT, axis=0, keepdims=True)
    zdot = jnp.sum(ztT * ptT, axis=0, keepdims=True)
    nx2 = jnp.sum(ztT * ztT, axis=0, keepdims=True)

    vel2 = vel2.reshape(BR, N)
    zdot = zdot.reshape(BR, N)
    nx2 = nx2.reshape(BR, N)

    vel = jnp.sqrt(vel2)
    nx = jnp.sqrt(nx2)
    ny = ny_ref[...]
    cos = zdot / jnp.maximum(nx * ny, 1e-8)
    surprise = jnp.clip(1.0 - cos, 0.0, 2.0) / 2.0
    sal = VEL_W * vel + SUR_W * surprise

    mn = jnp.min(sal, axis=-1, keepdims=True)
    mx = jnp.max(sal, axis=-1, keepdims=True)
    saln = (sal - mn) / (mx - mn + 1e-8)
    bsum = jnp.sum(saln, axis=0, keepdims=True)

    @pl.when(i == 0)
    def _init():
        acc_ref[...] = bsum

    @pl.when(i != 0)
    def _accum():
        acc_ref[...] = acc_ref[...] + bsum

    @pl.when(i == STEPS - 1)
    def _finish():
        acc = acc_ref[...]
        m = jnp.max(acc)
        eid = jax.lax.broadcasted_iota(jnp.int32, (1, N), 1)
        idx_ref[0, 0] = jnp.min(jnp.where(acc == m, eid, jnp.int32(2**30)))


def _scatter_body(idx_ref, mt_ref, y_any, o_any, src_ref, sem):
    del y_any
    n = idx_ref[0, 0]
    src_ref[...] = jnp.broadcast_to(mt_ref[...], (B, D))
    o3 = o_any.reshape(B, N, D)
    cp = pltpu.make_async_copy(src_ref, o3.at[:, n, :], sem)
    cp.start()
    cp.wait()


def kernel(z_t, z_tm1, prior, mask_token):
    z2 = z_t.reshape(B * N, D)
    zm2 = z_tm1.reshape(B * N, D)

    out_copy, idx = pl.pallas_call(
        _salience_body,
        grid=(STEPS,),
        in_specs=[
            pl.BlockSpec((RWS, D), lambda i: (i, 0)),
            pl.BlockSpec((RWS, D), lambda i: (i, 0)),
            pl.BlockSpec((N, D), lambda i: (0, 0)),
        ],
        out_specs=[
            pl.BlockSpec((RWS, D), lambda i: (i, 0)),
            pl.BlockSpec(memory_space=pltpu.SMEM),
        ],
        out_shape=[
            jax.ShapeDtypeStruct((B * N, D), jnp.float32),
            jax.ShapeDtypeStruct((1, 1), jnp.int32),
        ],
        scratch_shapes=[
            pltpu.VMEM((1, N), jnp.float32),
            pltpu.VMEM((D, RWS), jnp.float32),
            pltpu.VMEM((1, N), jnp.float32),
        ],
    )(z2, zm2, prior)

    mt2 = mask_token.reshape(1, D)

    masked = pl.pallas_call(
        _scatter_body,
        in_specs=[
            pl.BlockSpec(memory_space=pltpu.SMEM),
            pl.BlockSpec((1, D), lambda: (0, 0)),
            pl.BlockSpec(memory_space=pl.ANY),
        ],
        out_specs=pl.BlockSpec(memory_space=pl.ANY),
        out_shape=jax.ShapeDtypeStruct((B * N, D), jnp.float32),
        scratch_shapes=[
            pltpu.VMEM((B, D), jnp.float32),
            pltpu.SemaphoreType.DMA,
        ],
        input_output_aliases={2: 0},
    )(idx, mt2, out_copy)

    return masked.reshape(B, N, D)
